# baseline (speedup 1.0000x reference)
import dataclasses

import jax
import jax.numpy as jnp
from jax import lax
from jax.experimental import pallas as pl
from jax.experimental.pallas import tpu as pltpu
from jax.experimental.pallas import tpu_sc as plsc

BATCH = 16384
F = 16
NC = 2
NS = 16
NW = NC * NS
ROWS_PER_W = BATCH // NW
IDX_CHUNK = 128
N_CHUNKS = ROWS_PER_W // IDX_CHUNK


def _gmf_sc(user2d, item2d, user_emb, item_emb, params):
    mesh = plsc.VectorSubcoreMesh(core_axis_name="c", subcore_axis_name="s")
    cp = pltpu.CompilerParams(use_tc_tiling_on_sc=False)
    if "needs_layout_passes" in pltpu.CompilerParams.__dataclass_fields__:
        cp = dataclasses.replace(cp, needs_layout_passes=False)

    @pl.kernel(
        compiler_params=cp,
        out_type=jax.ShapeDtypeStruct((BATCH,), jnp.float32),
        mesh=mesh,
        scratch_types=[
            pltpu.VMEM((N_CHUNKS, IDX_CHUNK), jnp.int32),
            pltpu.VMEM((N_CHUNKS, IDX_CHUNK), jnp.int32),
            pltpu.VMEM((ROWS_PER_W, F), jnp.float32),
            pltpu.VMEM((ROWS_PER_W, F), jnp.float32),
            pltpu.VMEM((ROWS_PER_W,), jnp.float32),
            pltpu.VMEM((2, F), jnp.float32),
            pltpu.SemaphoreType.DMA,
        ],
    )
    def k(user_hbm, item_hbm, ue_hbm, ie_hbm, par_hbm, out_hbm,
          idx_u, idx_i, u_v, i_v, out_v, par_v, sem):
        wid = lax.axis_index("s") * NC + lax.axis_index("c")
        pltpu.sync_copy(user_hbm.at[pl.ds(wid * N_CHUNKS, N_CHUNKS), :], idx_u)
        pltpu.sync_copy(item_hbm.at[pl.ds(wid * N_CHUNKS, N_CHUNKS), :], idx_i)
        pltpu.sync_copy(par_hbm, par_v)
        copies = []
        for kk in range(N_CHUNKS):
            copies.append(pltpu.async_copy(
                ue_hbm.at[idx_u.at[kk]],
                u_v.at[pl.ds(kk * IDX_CHUNK, IDX_CHUNK), :], sem))
            copies.append(pltpu.async_copy(
                ie_hbm.at[idx_i.at[kk]],
                i_v.at[pl.ds(kk * IDX_CHUNK, IDX_CHUNK), :], sem))
        for cp in copies:
            cp.wait()

        w = par_v[0]
        bvec = par_v[1]
        lanes = lax.iota(jnp.int32, F)

        @pl.loop(0, ROWS_PER_W, step=F)
        def _(c):
            acc = bvec
            for r in range(F):
                t = u_v[c + r] * i_v[c + r] * w
                s = jnp.sum(t)
                acc = jnp.where(lanes == r, acc + s, acc)
            out_v[pl.ds(c, F)] = acc

        pltpu.sync_copy(out_v, out_hbm.at[pl.ds(wid * ROWS_PER_W, ROWS_PER_W)])

    return k(user2d, item2d, user_emb, item_emb, params)


@jax.jit
def kernel(user, item, user_emb, item_emb, W, b):
    user2d = user.astype(jnp.int32).reshape(NW * N_CHUNKS, IDX_CHUNK)
    item2d = item.astype(jnp.int32).reshape(NW * N_CHUNKS, IDX_CHUNK)
    params = jnp.concatenate(
        [W.reshape(1, F), jnp.broadcast_to(b.reshape(1, 1), (1, F))], axis=0)
    return _gmf_sc(user2d, item2d, user_emb, item_emb, params)
